# baseline (speedup 1.0000x reference)
import functools

import jax
import jax.numpy as jnp
from jax import lax
from jax.experimental import pallas as pl
from jax.experimental.pallas import tpu as pltpu
from jax.experimental.pallas import tpu_sc as plsc


@functools.cache
def _build(B: int, T: int):
    info = plsc.get_sparse_core_info()
    NC, NS, L = info.num_cores, info.num_subcores, info.num_lanes
    NW = NC * NS
    assert B % (8 * NW) == 0 and (B // NW) % L == 0 and T % 8 == 0
    b_per_w = B // NW

    mesh = plsc.VectorSubcoreMesh(core_axis_name="c", subcore_axis_name="s")

    @functools.partial(
        pl.kernel,
        mesh=mesh,
        out_type=jax.ShapeDtypeStruct((2 * B,), jnp.float32),
        compiler_params=pltpu.CompilerParams(needs_layout_passes=False),
        scratch_types=[
            pltpu.VMEM((b_per_w,), jnp.int32),
            pltpu.VMEM((2 * T,), jnp.float32),
            pltpu.VMEM((2 * b_per_w,), jnp.float32),
            pltpu.SemaphoreType.DMA,
        ],
    )
    def gather2(t_hbm, tab1_hbm, tab2_hbm, out_hbm, idx_v, tab_v, out_v, sem):
        wid = lax.axis_index("s") * NC + lax.axis_index("c")
        base = wid * b_per_w
        cp_idx = pltpu.make_async_copy(t_hbm.at[pl.ds(base, b_per_w)], idx_v, sem)
        cp_t1 = pltpu.make_async_copy(tab1_hbm, tab_v.at[pl.ds(0, T)], sem)
        cp_t2 = pltpu.make_async_copy(tab2_hbm, tab_v.at[pl.ds(T, T)], sem)
        cp_idx.start()
        cp_t1.start()
        cp_t2.start()
        cp_idx.wait()
        cp_t1.wait()
        cp_t2.wait()
        def body1(j, carry):
            o = j * L
            out_v[pl.ds(o, L)] = plsc.load_gather(tab_v, [idx_v[pl.ds(o, L)]])
            return carry

        def body2(j, carry):
            o = j * L
            out_v[pl.ds(b_per_w + o, L)] = plsc.load_gather(
                tab_v, [idx_v[pl.ds(o, L)] + T]
            )
            return carry

        lax.fori_loop(0, b_per_w // L, body1, 0, unroll=1)
        cp_o1 = pltpu.make_async_copy(
            out_v.at[pl.ds(0, b_per_w)], out_hbm.at[pl.ds(base, b_per_w)], sem
        )
        cp_o1.start()
        lax.fori_loop(0, b_per_w // L, body2, 0, unroll=1)
        cp_o2 = pltpu.make_async_copy(
            out_v.at[pl.ds(b_per_w, b_per_w)],
            out_hbm.at[pl.ds(B + base, b_per_w)],
            sem,
        )
        cp_o2.start()
        cp_o1.wait()
        cp_o2.wait()

    return gather2


def kernel(t, sqrt_alphas_cumprod, sqrt_one_minus_alphas_cumprod):
    B = t.shape[0]
    T = sqrt_alphas_cumprod.shape[0]
    planes = _build(B, T)(
        t.astype(jnp.int32),
        sqrt_alphas_cumprod.astype(jnp.float32),
        sqrt_one_minus_alphas_cumprod.astype(jnp.float32),
    )
    return planes.reshape(2, B, 1, 1).transpose(1, 0, 2, 3)


# speedup vs baseline: 1.0101x; 1.0101x over previous
import functools

import jax
import jax.numpy as jnp
from jax import lax
from jax.experimental import pallas as pl
from jax.experimental.pallas import tpu as pltpu
from jax.experimental.pallas import tpu_sc as plsc


@functools.cache
def _build(B: int, T: int):
    info = plsc.get_sparse_core_info()
    NC, NS, L = info.num_cores, info.num_subcores, info.num_lanes
    NW = NC * NS
    assert B % (8 * NW) == 0 and (B // NW) % L == 0 and T % 8 == 0
    b_per_w = B // NW

    mesh = plsc.VectorSubcoreMesh(core_axis_name="c", subcore_axis_name="s")

    @functools.partial(
        pl.kernel,
        mesh=mesh,
        out_type=jax.ShapeDtypeStruct((2 * B,), jnp.float32),
        compiler_params=pltpu.CompilerParams(needs_layout_passes=False),
        scratch_types=[
            pltpu.VMEM((b_per_w,), jnp.int32),
            pltpu.VMEM((2 * T,), jnp.float32),
            pltpu.VMEM((2 * b_per_w,), jnp.float32),
            pltpu.SemaphoreType.DMA,
        ],
    )
    def gather2(t_hbm, tab1_hbm, tab2_hbm, out_hbm, idx_v, tab_v, out_v, sem):
        wid = lax.axis_index("s") * NC + lax.axis_index("c")
        base = wid * b_per_w
        cp_idx = pltpu.make_async_copy(t_hbm.at[pl.ds(base, b_per_w)], idx_v, sem)
        cp_t1 = pltpu.make_async_copy(tab1_hbm, tab_v.at[pl.ds(0, T)], sem)
        cp_t2 = pltpu.make_async_copy(tab2_hbm, tab_v.at[pl.ds(T, T)], sem)
        cp_idx.start()
        cp_t1.start()
        cp_t2.start()
        cp_idx.wait()
        cp_t1.wait()
        cp_t2.wait()
        def body(j, carry):
            o = j * L
            idx = idx_v[pl.ds(o, L)]
            out_v[pl.ds(o, L)] = plsc.load_gather(tab_v, [idx])
            out_v[pl.ds(b_per_w + o, L)] = plsc.load_gather(tab_v, [idx + T])
            return carry

        lax.fori_loop(0, b_per_w // L, body, 0, unroll=2)
        cp_o1 = pltpu.make_async_copy(
            out_v.at[pl.ds(0, b_per_w)], out_hbm.at[pl.ds(base, b_per_w)], sem
        )
        cp_o2 = pltpu.make_async_copy(
            out_v.at[pl.ds(b_per_w, b_per_w)],
            out_hbm.at[pl.ds(B + base, b_per_w)],
            sem,
        )
        cp_o1.start()
        cp_o2.start()
        cp_o1.wait()
        cp_o2.wait()

    return gather2


def kernel(t, sqrt_alphas_cumprod, sqrt_one_minus_alphas_cumprod):
    B = t.shape[0]
    T = sqrt_alphas_cumprod.shape[0]
    planes = _build(B, T)(
        t.astype(jnp.int32),
        sqrt_alphas_cumprod.astype(jnp.float32),
        sqrt_one_minus_alphas_cumprod.astype(jnp.float32),
    )
    return planes.reshape(2, B, 1, 1).transpose(1, 0, 2, 3)
